# Initial kernel scaffold; baseline (speedup 1.0000x reference)
import jax
import jax.numpy as jnp
from jax.experimental import pallas as pl


def kernel(edge_index, n_id_cell, n_id_gene, logscale_cell, bias_cell, std_cell, logscale_gene, bias_gene, std_gene):
    raise NotImplementedError("write your pallas kernel here")


# speedup vs baseline: 234.7703x; 234.7703x over previous
import functools

import jax
import jax.numpy as jnp
from jax import lax
from jax.experimental import pallas as pl
from jax.experimental.pallas import tpu as pltpu
from jax.experimental.pallas import tpu_sc as plsc

NUM_EDGES = 3_200_000
NC, NS = 2, 16
NW = NC * NS
PER_W = NUM_EDGES // NW
CHUNK = 10_000
NCHUNK = PER_W // CHUNK


def _make_kernel():
    mesh = plsc.VectorSubcoreMesh(core_axis_name="c", subcore_axis_name="s")
    out_t = tuple(jax.ShapeDtypeStruct((NUM_EDGES,), jnp.float32) for _ in range(6))

    @functools.partial(
        pl.kernel,
        mesh=mesh,
        out_type=out_t,
        scratch_types=[
            pltpu.VMEM((CHUNK,), jnp.int32),
            pltpu.VMEM((CHUNK,), jnp.int32),
            pltpu.VMEM((CHUNK,), jnp.int32),
            pltpu.VMEM((CHUNK,), jnp.int32),
            pltpu.VMEM((CHUNK,), jnp.float32),
            pltpu.VMEM((CHUNK,), jnp.float32),
            pltpu.VMEM((CHUNK,), jnp.float32),
            pltpu.VMEM((CHUNK,), jnp.float32),
            pltpu.VMEM((CHUNK,), jnp.float32),
            pltpu.VMEM((CHUNK,), jnp.float32),
            pltpu.SemaphoreType.DMA,
            pltpu.SemaphoreType.DMA,
        ],
    )
    def run(ei, nid_c, nid_g, ls_c, bi_c, st_c, ls_g, bi_g, st_g,
            o_sls, o_sb, o_ss, o_dls, o_db, o_ds,
            i0_v, i1_v, n0_v, n1_v, sls_v, sb_v, ss_v, dls_v, db_v, ds_v,
            sem0, sem1):
        wid = lax.axis_index("s") * NC + lax.axis_index("c")

        def body(t, carry):
            base = wid * PER_W + t * CHUNK
            pltpu.sync_copy(ei.at[pl.ds(base, CHUNK)], i0_v)
            pltpu.sync_copy(ei.at[pl.ds(NUM_EDGES + base, CHUNK)], i1_v)
            h0 = pltpu.async_copy(nid_c.at[i0_v], n0_v, sem0)
            h1 = pltpu.async_copy(nid_g.at[i1_v], n1_v, sem1)
            h0.wait()
            h1.wait()
            g0 = pltpu.async_copy(ls_c.at[n0_v], sls_v, sem0)
            g1 = pltpu.async_copy(bi_c.at[n0_v], sb_v, sem0)
            g2 = pltpu.async_copy(st_c.at[n0_v], ss_v, sem0)
            g3 = pltpu.async_copy(ls_g.at[n1_v], dls_v, sem1)
            g4 = pltpu.async_copy(bi_g.at[n1_v], db_v, sem1)
            g5 = pltpu.async_copy(st_g.at[n1_v], ds_v, sem1)
            g0.wait()
            g1.wait()
            g2.wait()
            g3.wait()
            g4.wait()
            g5.wait()
            pltpu.sync_copy(sls_v, o_sls.at[pl.ds(base, CHUNK)])
            pltpu.sync_copy(sb_v, o_sb.at[pl.ds(base, CHUNK)])
            pltpu.sync_copy(ss_v, o_ss.at[pl.ds(base, CHUNK)])
            pltpu.sync_copy(dls_v, o_dls.at[pl.ds(base, CHUNK)])
            pltpu.sync_copy(db_v, o_db.at[pl.ds(base, CHUNK)])
            pltpu.sync_copy(ds_v, o_ds.at[pl.ds(base, CHUNK)])
            return carry

        lax.fori_loop(0, NCHUNK, body, 0)

    return run


_RUN = _make_kernel()


def kernel(edge_index, n_id_cell, n_id_gene, logscale_cell, bias_cell,
           std_cell, logscale_gene, bias_gene, std_gene):
    return _RUN(edge_index.reshape(-1), n_id_cell, n_id_gene, logscale_cell,
                bias_cell, std_cell, logscale_gene, bias_gene, std_gene)


# speedup vs baseline: 280.8805x; 1.1964x over previous
import functools

import jax
import jax.numpy as jnp
from jax import lax
from jax.experimental import pallas as pl
from jax.experimental.pallas import tpu as pltpu
from jax.experimental.pallas import tpu_sc as plsc

NUM_EDGES = 3_200_000
NC, NS, L = 2, 16, 16
NW = NC * NS
PER_W = NUM_EDGES // NW
CHUNK = 4_000
NCHUNK = PER_W // CHUNK
NVEC = CHUNK // L


def _make_kernel():
    mesh = plsc.VectorSubcoreMesh(core_axis_name="c", subcore_axis_name="s")
    out_t = tuple(jax.ShapeDtypeStruct((NUM_EDGES,), jnp.float32) for _ in range(6))

    @functools.partial(
        pl.kernel,
        mesh=mesh,
        out_type=out_t,
        compiler_params=pltpu.CompilerParams(
            needs_layout_passes=False, use_tc_tiling_on_sc=False),
        scratch_types=[
            pltpu.VMEM((CHUNK,), jnp.int32),
            pltpu.VMEM((CHUNK,), jnp.int32),
            pltpu.VMEM((CHUNK,), jnp.int32),
            pltpu.VMEM((CHUNK,), jnp.int32),
            pltpu.VMEM((CHUNK, 8), jnp.float32),
            pltpu.VMEM((CHUNK, 8), jnp.float32),
            pltpu.VMEM((CHUNK,), jnp.float32),
            pltpu.VMEM((CHUNK,), jnp.float32),
            pltpu.VMEM((CHUNK,), jnp.float32),
            pltpu.VMEM((CHUNK,), jnp.float32),
            pltpu.VMEM((CHUNK,), jnp.float32),
            pltpu.VMEM((CHUNK,), jnp.float32),
            pltpu.SemaphoreType.DMA,
            pltpu.SemaphoreType.DMA,
        ],
    )
    def run(ei, nid_c, nid_g, p_c, p_g,
            o_sls, o_sb, o_ss, o_dls, o_db, o_ds,
            i0_v, i1_v, n0_v, n1_v, r0_v, r1_v,
            sls_v, sb_v, ss_v, dls_v, db_v, ds_v,
            sem0, sem1):
        wid = lax.axis_index("s") * NC + lax.axis_index("c")

        def body(t, carry):
            base = wid * PER_W + t * CHUNK
            pltpu.sync_copy(ei.at[pl.ds(base, CHUNK)], i0_v)
            pltpu.sync_copy(ei.at[pl.ds(NUM_EDGES + base, CHUNK)], i1_v)
            h0 = pltpu.async_copy(nid_c.at[i0_v], n0_v, sem0)
            h1 = pltpu.async_copy(nid_g.at[i1_v], n1_v, sem1)
            h0.wait()
            h1.wait()
            g0 = pltpu.async_copy(p_c.at[n0_v], r0_v, sem0)
            g1 = pltpu.async_copy(p_g.at[n1_v], r1_v, sem1)
            g0.wait()
            g1.wait()

            def unpack(i, c2):
                row = i * L + lax.iota(jnp.int32, L)
                for rows, bufs in ((r0_v, (sls_v, sb_v, ss_v)),
                                   (r1_v, (dls_v, db_v, ds_v))):
                    for col, buf in enumerate(bufs):
                        cvec = jnp.full((L,), col, jnp.int32)
                        buf[pl.ds(i * L, L)] = plsc.load_gather(rows, [row, cvec])
                return c2

            lax.fori_loop(0, NVEC, unpack, 0)

            pltpu.sync_copy(sls_v, o_sls.at[pl.ds(base, CHUNK)])
            pltpu.sync_copy(sb_v, o_sb.at[pl.ds(base, CHUNK)])
            pltpu.sync_copy(ss_v, o_ss.at[pl.ds(base, CHUNK)])
            pltpu.sync_copy(dls_v, o_dls.at[pl.ds(base, CHUNK)])
            pltpu.sync_copy(db_v, o_db.at[pl.ds(base, CHUNK)])
            pltpu.sync_copy(ds_v, o_ds.at[pl.ds(base, CHUNK)])
            return carry

        lax.fori_loop(0, NCHUNK, body, 0)

    return run


_RUN = _make_kernel()


def kernel(edge_index, n_id_cell, n_id_gene, logscale_cell, bias_cell,
           std_cell, logscale_gene, bias_gene, std_gene):
    zc = jnp.zeros_like(logscale_cell)
    zg = jnp.zeros_like(logscale_gene)
    p_cell = jnp.stack(
        [logscale_cell, bias_cell, std_cell, zc, zc, zc, zc, zc], axis=1)
    p_gene = jnp.stack(
        [logscale_gene, bias_gene, std_gene, zg, zg, zg, zg, zg], axis=1)
    return _RUN(edge_index.reshape(-1), n_id_cell, n_id_gene, p_cell, p_gene)


# speedup vs baseline: 353.8744x; 1.2599x over previous
import functools

import jax
import jax.numpy as jnp
from jax import lax
from jax.experimental import pallas as pl
from jax.experimental.pallas import tpu as pltpu
from jax.experimental.pallas import tpu_sc as plsc

NUM_EDGES = 3_200_000
NUM_NODES = 100_000
NC, NS, L = 2, 16, 16
NW = NC * NS
PER_W = NUM_EDGES // NW
CHUNK = 4_000
NCHUNK = PER_W // CHUNK
NVEC = CHUNK // L

PRE_CHUNK = 1_000
PRE_NCHUNK = NUM_NODES // PRE_CHUNK

_SC_PARAMS = pltpu.CompilerParams(
    needs_layout_passes=False, use_tc_tiling_on_sc=False)


def _make_precompose():
    mesh = plsc.VectorSubcoreMesh(core_axis_name="c", subcore_axis_name="s")
    out_t = tuple(
        jax.ShapeDtypeStruct((NUM_NODES, 8), jnp.float32) for _ in range(2))

    @functools.partial(
        pl.kernel,
        mesh=mesh,
        out_type=out_t,
        compiler_params=_SC_PARAMS,
        scratch_types=[
            pltpu.VMEM((PRE_CHUNK,), jnp.int32),
            pltpu.VMEM((PRE_CHUNK, 8), jnp.float32),
            pltpu.SemaphoreType.DMA,
        ],
    )
    def run(nid_c, nid_g, p_c, p_g, c_c, c_g, nid_v, rows_v, sem):
        wid = lax.axis_index("s") * NC + lax.axis_index("c")

        def body(k, carry):
            cid = wid + NW * k

            @pl.when(cid < PRE_NCHUNK)
            def _():
                base = cid * PRE_CHUNK
                for nid, p, c in ((nid_c, p_c, c_c), (nid_g, p_g, c_g)):
                    pltpu.sync_copy(nid.at[pl.ds(base, PRE_CHUNK)], nid_v)
                    pltpu.async_copy(p.at[nid_v], rows_v, sem).wait()
                    pltpu.sync_copy(rows_v, c.at[pl.ds(base, PRE_CHUNK), :])

            return carry

        lax.fori_loop(0, (PRE_NCHUNK + NW - 1) // NW, body, 0)

    return run


def _make_main():
    mesh = plsc.VectorSubcoreMesh(core_axis_name="c", subcore_axis_name="s")
    out_t = tuple(
        jax.ShapeDtypeStruct((NUM_EDGES,), jnp.float32) for _ in range(6))

    @functools.partial(
        pl.kernel,
        mesh=mesh,
        out_type=out_t,
        compiler_params=_SC_PARAMS,
        scratch_types=[
            pltpu.VMEM((CHUNK,), jnp.int32),
            pltpu.VMEM((CHUNK,), jnp.int32),
            pltpu.VMEM((CHUNK, 8), jnp.float32),
            pltpu.VMEM((CHUNK, 8), jnp.float32),
            pltpu.VMEM((CHUNK,), jnp.float32),
            pltpu.VMEM((CHUNK,), jnp.float32),
            pltpu.VMEM((CHUNK,), jnp.float32),
            pltpu.VMEM((CHUNK,), jnp.float32),
            pltpu.VMEM((CHUNK,), jnp.float32),
            pltpu.VMEM((CHUNK,), jnp.float32),
            pltpu.SemaphoreType.DMA,
            pltpu.SemaphoreType.DMA,
        ],
    )
    def run(ei, c_c, c_g,
            o_sls, o_sb, o_ss, o_dls, o_db, o_ds,
            i0_v, i1_v, r0_v, r1_v,
            sls_v, sb_v, ss_v, dls_v, db_v, ds_v,
            sem0, sem1):
        wid = lax.axis_index("s") * NC + lax.axis_index("c")

        def body(t, carry):
            base = wid * PER_W + t * CHUNK
            pltpu.sync_copy(ei.at[pl.ds(base, CHUNK)], i0_v)
            pltpu.sync_copy(ei.at[pl.ds(NUM_EDGES + base, CHUNK)], i1_v)
            g0 = pltpu.async_copy(c_c.at[i0_v], r0_v, sem0)
            g1 = pltpu.async_copy(c_g.at[i1_v], r1_v, sem1)
            g0.wait()
            g1.wait()

            def unpack(i, c2):
                row = i * L + lax.iota(jnp.int32, L)
                for rows, bufs in ((r0_v, (sls_v, sb_v, ss_v)),
                                   (r1_v, (dls_v, db_v, ds_v))):
                    for col, buf in enumerate(bufs):
                        cvec = jnp.full((L,), col, jnp.int32)
                        buf[pl.ds(i * L, L)] = plsc.load_gather(rows, [row, cvec])
                return c2

            lax.fori_loop(0, NVEC, unpack, 0)

            pltpu.sync_copy(sls_v, o_sls.at[pl.ds(base, CHUNK)])
            pltpu.sync_copy(sb_v, o_sb.at[pl.ds(base, CHUNK)])
            pltpu.sync_copy(ss_v, o_ss.at[pl.ds(base, CHUNK)])
            pltpu.sync_copy(dls_v, o_dls.at[pl.ds(base, CHUNK)])
            pltpu.sync_copy(db_v, o_db.at[pl.ds(base, CHUNK)])
            pltpu.sync_copy(ds_v, o_ds.at[pl.ds(base, CHUNK)])
            return carry

        lax.fori_loop(0, NCHUNK, body, 0)

    return run


_PRE = _make_precompose()
_MAIN = _make_main()


def kernel(edge_index, n_id_cell, n_id_gene, logscale_cell, bias_cell,
           std_cell, logscale_gene, bias_gene, std_gene):
    zc = jnp.zeros_like(logscale_cell)
    zg = jnp.zeros_like(logscale_gene)
    p_cell = jnp.stack(
        [logscale_cell, bias_cell, std_cell, zc, zc, zc, zc, zc], axis=1)
    p_gene = jnp.stack(
        [logscale_gene, bias_gene, std_gene, zg, zg, zg, zg, zg], axis=1)
    c_cell, c_gene = _PRE(n_id_cell, n_id_gene, p_cell, p_gene)
    return _MAIN(edge_index.reshape(-1), c_cell, c_gene)


# speedup vs baseline: 476.3304x; 1.3460x over previous
import functools

import jax
import jax.numpy as jnp
from jax import lax
from jax.experimental import pallas as pl
from jax.experimental.pallas import tpu as pltpu
from jax.experimental.pallas import tpu_sc as plsc

NUM_EDGES = 3_200_000
NUM_NODES = 100_000
NC, NS, L = 2, 16, 16
NW = NC * NS
PER_W = NUM_EDGES // NW
CHUNK = 2_000
NCHUNK = PER_W // CHUNK
NVEC = CHUNK // L

PRE_CHUNK = 1_000
PRE_NCHUNK = NUM_NODES // PRE_CHUNK

_SC_PARAMS = pltpu.CompilerParams(
    needs_layout_passes=False, use_tc_tiling_on_sc=False)


def _make_precompose():
    mesh = plsc.VectorSubcoreMesh(core_axis_name="c", subcore_axis_name="s")
    out_t = tuple(
        jax.ShapeDtypeStruct((NUM_NODES, 8), jnp.float32) for _ in range(2))

    @functools.partial(
        pl.kernel,
        mesh=mesh,
        out_type=out_t,
        compiler_params=_SC_PARAMS,
        scratch_types=[
            pltpu.VMEM((PRE_CHUNK,), jnp.int32),
            pltpu.VMEM((PRE_CHUNK, 8), jnp.float32),
            pltpu.SemaphoreType.DMA,
        ],
    )
    def run(nid_c, nid_g, p_c, p_g, c_c, c_g, nid_v, rows_v, sem):
        wid = lax.axis_index("s") * NC + lax.axis_index("c")

        def body(k, carry):
            cid = wid + NW * k

            @pl.when(cid < PRE_NCHUNK)
            def _():
                base = cid * PRE_CHUNK
                for nid, p, c in ((nid_c, p_c, c_c), (nid_g, p_g, c_g)):
                    pltpu.sync_copy(nid.at[pl.ds(base, PRE_CHUNK)], nid_v)
                    pltpu.async_copy(p.at[nid_v], rows_v, sem).wait()
                    pltpu.sync_copy(rows_v, c.at[pl.ds(base, PRE_CHUNK), :])

            return carry

        lax.fori_loop(0, (PRE_NCHUNK + NW - 1) // NW, body, 0)

    return run


def _make_main():
    mesh = plsc.VectorSubcoreMesh(core_axis_name="c", subcore_axis_name="s")
    out_t = tuple(
        jax.ShapeDtypeStruct((NUM_EDGES,), jnp.float32) for _ in range(6))

    @functools.partial(
        pl.kernel,
        mesh=mesh,
        out_type=out_t,
        compiler_params=_SC_PARAMS,
        scratch_types=[
            pltpu.VMEM((CHUNK,), jnp.int32),
            pltpu.VMEM((CHUNK,), jnp.int32),
            pltpu.VMEM((CHUNK,), jnp.int32),
            pltpu.VMEM((CHUNK,), jnp.int32),
            pltpu.VMEM((CHUNK, 8), jnp.float32),
            pltpu.VMEM((CHUNK, 8), jnp.float32),
            pltpu.VMEM((CHUNK, 8), jnp.float32),
            pltpu.VMEM((CHUNK, 8), jnp.float32),
            pltpu.VMEM((CHUNK,), jnp.float32),
            pltpu.VMEM((CHUNK,), jnp.float32),
            pltpu.VMEM((CHUNK,), jnp.float32),
            pltpu.VMEM((CHUNK,), jnp.float32),
            pltpu.VMEM((CHUNK,), jnp.float32),
            pltpu.VMEM((CHUNK,), jnp.float32),
            pltpu.SemaphoreType.DMA,
            pltpu.SemaphoreType.DMA,
            pltpu.SemaphoreType.DMA,
            pltpu.SemaphoreType.DMA,
        ],
    )
    def run(ei, c_c, c_g,
            o_sls, o_sb, o_ss, o_dls, o_db, o_ds,
            i0a_v, i1a_v, i0b_v, i1b_v, r0a_v, r1a_v, r0b_v, r1b_v,
            sls_v, sb_v, ss_v, dls_v, db_v, ds_v,
            sa0, sa1, sb0, sb1):
        wid = lax.axis_index("s") * NC + lax.axis_index("c")
        bufs = ((i0a_v, i1a_v, r0a_v, r1a_v, sa0, sa1),
                (i0b_v, i1b_v, r0b_v, r1b_v, sb0, sb1))

        def load_and_issue(t, which):
            i0_v, i1_v, r0_v, r1_v, s0, s1 = bufs[which]
            base = wid * PER_W + t * CHUNK
            pltpu.sync_copy(ei.at[pl.ds(base, CHUNK)], i0_v)
            pltpu.sync_copy(ei.at[pl.ds(NUM_EDGES + base, CHUNK)], i1_v)
            pltpu.async_copy(c_c.at[i0_v], r0_v, s0)
            pltpu.async_copy(c_g.at[i1_v], r1_v, s1)

        def drain_unpack_store(t, which):
            i0_v, i1_v, r0_v, r1_v, s0, s1 = bufs[which]
            base = wid * PER_W + t * CHUNK
            pltpu.make_async_copy(c_c.at[i0_v], r0_v, s0).wait()
            pltpu.make_async_copy(c_g.at[i1_v], r1_v, s1).wait()

            def unpack(i, c2):
                row = i * L + lax.iota(jnp.int32, L)
                for rows, obufs in ((r0_v, (sls_v, sb_v, ss_v)),
                                    (r1_v, (dls_v, db_v, ds_v))):
                    for col, buf in enumerate(obufs):
                        cvec = jnp.full((L,), col, jnp.int32)
                        buf[pl.ds(i * L, L)] = plsc.load_gather(rows, [row, cvec])
                return c2

            lax.fori_loop(0, NVEC, unpack, 0)

            pltpu.sync_copy(sls_v, o_sls.at[pl.ds(base, CHUNK)])
            pltpu.sync_copy(sb_v, o_sb.at[pl.ds(base, CHUNK)])
            pltpu.sync_copy(ss_v, o_ss.at[pl.ds(base, CHUNK)])
            pltpu.sync_copy(dls_v, o_dls.at[pl.ds(base, CHUNK)])
            pltpu.sync_copy(db_v, o_db.at[pl.ds(base, CHUNK)])
            pltpu.sync_copy(ds_v, o_ds.at[pl.ds(base, CHUNK)])

        load_and_issue(0, 0)

        @pl.loop(0, NCHUNK, step=2)
        def _(t):
            @pl.when(t + 1 < NCHUNK)
            def _():
                load_and_issue(t + 1, 1)

            drain_unpack_store(t, 0)

            @pl.when(t + 2 < NCHUNK)
            def _():
                load_and_issue(t + 2, 0)

            @pl.when(t + 1 < NCHUNK)
            def _():
                drain_unpack_store(t + 1, 1)

    return run


_PRE = _make_precompose()
_MAIN = _make_main()


def kernel(edge_index, n_id_cell, n_id_gene, logscale_cell, bias_cell,
           std_cell, logscale_gene, bias_gene, std_gene):
    zc = jnp.zeros_like(logscale_cell)
    zg = jnp.zeros_like(logscale_gene)
    p_cell = jnp.stack(
        [logscale_cell, bias_cell, std_cell, zc, zc, zc, zc, zc], axis=1)
    p_gene = jnp.stack(
        [logscale_gene, bias_gene, std_gene, zg, zg, zg, zg, zg], axis=1)
    c_cell, c_gene = _PRE(n_id_cell, n_id_gene, p_cell, p_gene)
    return _MAIN(edge_index.reshape(-1), c_cell, c_gene)
